# Initial kernel scaffold; baseline (speedup 1.0000x reference)
import jax
import jax.numpy as jnp
from jax.experimental import pallas as pl


def kernel(inputTSDF, W1, b1, W2, b2, W3, b3, W4, b4):
    raise NotImplementedError("write your pallas kernel here")


# speedup vs baseline: 5.2805x; 5.2805x over previous
import functools

import jax
import jax.numpy as jnp
from jax.experimental import pallas as pl
from jax.experimental.pallas import tpu as pltpu

D = 64
HP = 66
LP = HP * HP
LI = LP - 2 * HP
LV = LP - 2 * HP - 2
TILE_D = 16
HALO = 4
SLAB = TILE_D + 2 * HALO

_OFF9 = (-67, -66, -65, -1, 0, 1, 65, 66, 67)


def _net_kernel(x_ref, w1_ref, b1_ref, w2_ref, b2_ref, w3_ref, b3_ref,
                w4_ref, b4_ref, o_ref,
                xn_ref, m2_ref, m3_ref, a1_ref, a2_ref, a3_ref):
    i = pl.program_id(1)

    for ref in (a1_ref, a2_ref, a3_ref):
        ref[:, :, 0:67] = jnp.zeros(ref.shape[:2] + (67,), jnp.float32)
        ref[:, :, LP - 67:LP] = jnp.zeros(ref.shape[:2] + (67,), jnp.float32)

    slab = x_ref[0, pl.ds(i * TILE_D, SLAB), :]
    mask = (jnp.abs(slab) != 0).astype(jnp.float32)
    xn_ref[...] = (slab / 2.0 + 0.5) * mask
    m2_ref[...] = mask
    for z in range(SLAB):
        m3_ref[z, :, :] = mask[z:z + 1, :]

    for z in range(1, SLAB - 1):
        x3 = xn_ref[z - 1:z + 2, :]
        xt = jnp.concatenate(
            [x3[:, 0:LI], x3[:, HP:HP + LI], x3[:, 2 * HP:2 * HP + LI]],
            axis=0)
        p = jnp.dot(w1_ref[...], xt, preferred_element_type=jnp.float32)
        y = p[0:16, 0:LV] + p[16:32, 1:1 + LV] + p[32:48, 2:2 + LV]
        m = m2_ref[z:z + 1, 67:67 + LV]
        a1_ref[z, :, pl.ds(67, LV)] = (
            jnp.maximum(y + b1_ref[...], 0.0) * m)

    def conv2(z, _):
        x3 = a1_ref[pl.ds(z - 1, 3)]
        xt = jnp.concatenate(
            [x3[:, :, 0:LI].reshape(48, LI),
             x3[:, :, HP:HP + LI].reshape(48, LI),
             x3[:, :, 2 * HP:2 * HP + LI].reshape(48, LI)],
            axis=0)
        p = jnp.dot(w2_ref[...], xt, preferred_element_type=jnp.float32)
        y = p[0:32, 0:LV] + p[32:64, 1:1 + LV] + p[64:96, 2:2 + LV]
        m = m3_ref[pl.ds(z, 1), :, 67:67 + LV].reshape(1, LV)
        y = jnp.maximum(y + b2_ref[...], 0.0) * m
        a2_ref[pl.ds(z, 1), :, pl.ds(67, LV)] = y.reshape(1, 32, LV)
        return 0

    def conv3(z, _):
        xt = a2_ref[pl.ds(z - 1, 3)].reshape(96, LP)
        p = jnp.dot(w3_ref[...], xt, preferred_element_type=jnp.float32)
        y = p[0:16, 67 + _OFF9[0]:67 + _OFF9[0] + LV]
        for t in range(1, 9):
            o = 67 + _OFF9[t]
            y = y + p[t * 16:(t + 1) * 16, o:o + LV]
        m = m3_ref[pl.ds(z, 1), :, 67:67 + LV].reshape(1, LV)
        y = jnp.maximum(y + b3_ref[...], 0.0) * m
        a3_ref[pl.ds(z, 1), :, pl.ds(67, LV)] = y.reshape(1, 16, LV)
        return 0

    jax.lax.fori_loop(2, SLAB - 2, conv2, 0)
    jax.lax.fori_loop(3, SLAB - 3, conv3, 0)

    zpad = jnp.zeros((1, 67), jnp.float32)
    for z in range(HALO, HALO + TILE_D):
        xt = a3_ref[z - 1:z + 2].reshape(48, LP)
        p = jnp.dot(w4_ref[...], xt, preferred_element_type=jnp.float32)
        y = p[0:1, 67 + _OFF9[0]:67 + _OFF9[0] + LV]
        for t in range(1, 9):
            o = 67 + _OFF9[t]
            y = y + p[t:t + 1, o:o + LV]
        m = m2_ref[z:z + 1, 67:67 + LV]
        y = jnp.maximum(y + b4_ref[...], 0.0) * m
        row = jnp.concatenate([zpad, y, zpad], axis=1) * 2.0 - 1.0
        o_ref[0, z - HALO, :] = row[0:1, :].reshape(LP)


def kernel(inputTSDF, W1, b1, W2, b2, W3, b3, W4, b4):
    n = inputTSDF.shape[0]
    x = inputTSDF[:, 0]
    xp = jnp.pad(x, ((0, 0), (HALO, HALO), (1, 1), (1, 1)))
    xp = xp.reshape(n, D + 2 * HALO, LP)

    wm1 = W1[:, 0].transpose(3, 0, 2, 1).reshape(48, 9)
    wm2 = W2.transpose(4, 0, 3, 2, 1).reshape(96, 144)
    wm3 = W3.transpose(3, 4, 0, 2, 1).reshape(144, 96)
    wm4 = W4[0].transpose(2, 3, 1, 0).reshape(9, 48)

    grid = (n, D // TILE_D)
    full = lambda b, i: (0, 0)
    out = pl.pallas_call(
        _net_kernel,
        grid=grid,
        in_specs=[
            pl.BlockSpec((1, D + 2 * HALO, LP), lambda b, i: (b, 0, 0)),
            pl.BlockSpec((48, 9), full),
            pl.BlockSpec((16, 1), full),
            pl.BlockSpec((96, 144), full),
            pl.BlockSpec((32, 1), full),
            pl.BlockSpec((144, 96), full),
            pl.BlockSpec((16, 1), full),
            pl.BlockSpec((9, 48), full),
            pl.BlockSpec((1, 1), full),
        ],
        out_specs=pl.BlockSpec((1, TILE_D, LP), lambda b, i: (b, i, 0)),
        out_shape=jax.ShapeDtypeStruct((n, D, LP), jnp.float32),
        scratch_shapes=[
            pltpu.VMEM((SLAB, LP), jnp.float32),
            pltpu.VMEM((SLAB, LP), jnp.float32),
            pltpu.VMEM((SLAB, 1, LP), jnp.float32),
            pltpu.VMEM((SLAB, 16, LP), jnp.float32),
            pltpu.VMEM((SLAB, 32, LP), jnp.float32),
            pltpu.VMEM((SLAB, 16, LP), jnp.float32),
        ],
    )(xp, wm1, b1.reshape(16, 1), wm2, b2.reshape(32, 1),
      wm3, b3.reshape(16, 1), wm4, b4.reshape(1, 1))
    out = out.reshape(n, D, HP, HP)[:, :, 1:65, 1:65]
    return out[:, None]


# speedup vs baseline: 6.0264x; 1.1413x over previous
import functools

import jax
import jax.numpy as jnp
from jax.experimental import pallas as pl
from jax.experimental.pallas import tpu as pltpu

D = 64
HP = 66
LP = HP * HP
LI = LP - 2 * HP
LV = LP - 2 * HP - 2
TILE_D = 16
HALO = 4
SLAB = TILE_D + 2 * HALO

_OFF9 = (-67, -66, -65, -1, 0, 1, 65, 66, 67)


def _net_kernel(x_ref, w1_ref, b1_ref, w2_ref, b2_ref, w3_ref, b3_ref,
                w4_ref, b4_ref, o_ref,
                xn_ref, m2_ref, m3_ref, a1_ref, a2_ref, a3_ref, p4_ref):
    i = pl.program_id(1)

    for ref in (a1_ref, a2_ref, a3_ref):
        ref[:, :, 0:67] = jnp.zeros(ref.shape[:2] + (67,), jnp.float32)
        ref[:, :, LP - 67:LP] = jnp.zeros(ref.shape[:2] + (67,), jnp.float32)

    slab = x_ref[0, pl.ds(i * TILE_D, SLAB), :]
    mask = (jnp.abs(slab) != 0).astype(jnp.float32)
    xn_ref[...] = (slab / 2.0 + 0.5) * mask
    m2_ref[...] = mask
    for z in range(SLAB):
        m3_ref[z, :, :] = mask[z:z + 1, :]

    for z in range(1, SLAB - 1):
        x3 = xn_ref[z - 1:z + 2, :]
        p = jnp.dot(w1_ref[:, 0:3], x3[:, 0:LI],
                    preferred_element_type=jnp.float32)
        p = p + jnp.dot(w1_ref[:, 3:6], x3[:, HP:HP + LI],
                        preferred_element_type=jnp.float32)
        p = p + jnp.dot(w1_ref[:, 6:9], x3[:, 2 * HP:2 * HP + LI],
                        preferred_element_type=jnp.float32)
        y = p[0:16, 0:LV] + p[16:32, 1:1 + LV] + p[32:48, 2:2 + LV]
        m = m2_ref[z:z + 1, 67:67 + LV]
        a1_ref[z, :, pl.ds(67, LV)] = (
            jnp.maximum(y + b1_ref[...], 0.0) * m)

    def conv2(z, _):
        x3 = a1_ref[pl.ds(z - 1, 3)]
        xt = jnp.concatenate(
            [x3[:, :, 0:LI].reshape(48, LI),
             x3[:, :, HP:HP + LI].reshape(48, LI),
             x3[:, :, 2 * HP:2 * HP + LI].reshape(48, LI)],
            axis=0)
        p = jnp.dot(w2_ref[...], xt, preferred_element_type=jnp.float32)
        y = p[0:32, 0:LV] + p[32:64, 1:1 + LV] + p[64:96, 2:2 + LV]
        m = m3_ref[pl.ds(z, 1), :, 67:67 + LV].reshape(1, LV)
        y = jnp.maximum(y + b2_ref[...], 0.0) * m
        a2_ref[pl.ds(z, 1), :, pl.ds(67, LV)] = y.reshape(1, 32, LV)
        return 0

    def conv3(z, _):
        xt = a2_ref[pl.ds(z - 1, 3)].reshape(96, LP)
        p = jnp.dot(w3_ref[...], xt, preferred_element_type=jnp.float32)
        y = p[0:16, 67 + _OFF9[0]:67 + _OFF9[0] + LV]
        for t in range(1, 9):
            o = 67 + _OFF9[t]
            y = y + p[t * 16:(t + 1) * 16, o:o + LV]
        m = m3_ref[pl.ds(z, 1), :, 67:67 + LV].reshape(1, LV)
        y = jnp.maximum(y + b3_ref[...], 0.0) * m
        a3_ref[pl.ds(z, 1), :, pl.ds(67, LV)] = y.reshape(1, 16, LV)
        return 0

    jax.lax.fori_loop(2, SLAB - 2, conv2, 0)
    jax.lax.fori_loop(3, SLAB - 3, conv3, 0)

    for z in range(HALO, HALO + TILE_D):
        xt = a3_ref[z - 1:z + 2].reshape(48, LP)
        p4_ref[z - HALO] = jnp.dot(
            w4_ref[...], xt, preferred_element_type=jnp.float32)
    y = p4_ref[:, 0, 67 + _OFF9[0]:67 + _OFF9[0] + LV]
    for t in range(1, 9):
        o = 67 + _OFF9[t]
        y = y + p4_ref[:, t, o:o + LV]
    m = m2_ref[HALO:HALO + TILE_D, 67:67 + LV]
    y = jnp.maximum(y + b4_ref[...], 0.0) * m
    zpad = jnp.zeros((TILE_D, 67), jnp.float32)
    o_ref[0] = jnp.concatenate([zpad, y, zpad], axis=1) * 2.0 - 1.0


def kernel(inputTSDF, W1, b1, W2, b2, W3, b3, W4, b4):
    n = inputTSDF.shape[0]
    x = inputTSDF[:, 0]
    xp = jnp.pad(x, ((0, 0), (HALO, HALO), (1, 1), (1, 1)))
    xp = xp.reshape(n, D + 2 * HALO, LP)

    wm1 = W1[:, 0].transpose(3, 0, 2, 1).reshape(48, 9)
    wm2 = W2.transpose(4, 0, 3, 2, 1).reshape(96, 144)
    wm3 = W3.transpose(3, 4, 0, 2, 1).reshape(144, 96)
    wm4 = W4[0].transpose(2, 3, 1, 0).reshape(9, 48)

    grid = (n, D // TILE_D)
    full = lambda b, i: (0, 0)
    out = pl.pallas_call(
        _net_kernel,
        grid=grid,
        in_specs=[
            pl.BlockSpec((1, D + 2 * HALO, LP), lambda b, i: (b, 0, 0)),
            pl.BlockSpec((48, 9), full),
            pl.BlockSpec((16, 1), full),
            pl.BlockSpec((96, 144), full),
            pl.BlockSpec((32, 1), full),
            pl.BlockSpec((144, 96), full),
            pl.BlockSpec((16, 1), full),
            pl.BlockSpec((9, 48), full),
            pl.BlockSpec((1, 1), full),
        ],
        out_specs=pl.BlockSpec((1, TILE_D, LP), lambda b, i: (b, i, 0)),
        out_shape=jax.ShapeDtypeStruct((n, D, LP), jnp.float32),
        scratch_shapes=[
            pltpu.VMEM((SLAB, LP), jnp.float32),
            pltpu.VMEM((SLAB, LP), jnp.float32),
            pltpu.VMEM((SLAB, 1, LP), jnp.float32),
            pltpu.VMEM((SLAB, 16, LP), jnp.float32),
            pltpu.VMEM((SLAB, 32, LP), jnp.float32),
            pltpu.VMEM((SLAB, 16, LP), jnp.float32),
            pltpu.VMEM((TILE_D, 9, LP), jnp.float32),
        ],
        compiler_params=pltpu.CompilerParams(
            dimension_semantics=("parallel", "parallel")),
    )(xp, wm1, b1.reshape(16, 1), wm2, b2.reshape(32, 1),
      wm3, b3.reshape(16, 1), wm4, b4.reshape(1, 1))
    out = out.reshape(n, D, HP, HP)[:, :, 1:65, 1:65]
    return out[:, None]


# speedup vs baseline: 6.0587x; 1.0054x over previous
import functools

import jax
import jax.numpy as jnp
from jax.experimental import pallas as pl
from jax.experimental.pallas import tpu as pltpu

D = 64
HP = 66
LP = HP * HP
LI = LP - 2 * HP
LV = LP - 2 * HP - 2
TILE_D = 16
HALO = 4
SLAB = TILE_D + 2 * HALO

_OFF9 = (-67, -66, -65, -1, 0, 1, 65, 66, 67)


def _net_kernel(x_ref, w1_ref, b1_ref, w2_ref, b2_ref, w3_ref, b3_ref,
                w4_ref, b4_ref, o_ref,
                xn_ref, m2_ref, m3_ref, a1_ref, a2_ref, a3_ref, p4_ref):
    i = pl.program_id(1)

    for ref in (a1_ref, a2_ref, a3_ref):
        ref[:, :, 0:67] = jnp.zeros(ref.shape[:2] + (67,), jnp.float32)
        ref[:, :, LP - 67:LP] = jnp.zeros(ref.shape[:2] + (67,), jnp.float32)

    slab = x_ref[0, pl.ds(i * TILE_D, SLAB), :]
    mask = (jnp.abs(slab) != 0).astype(jnp.float32)
    xn_ref[...] = (slab / 2.0 + 0.5) * mask
    m2_ref[...] = mask
    for z in range(SLAB):
        m3_ref[z, :, :] = mask[z:z + 1, :]

    for z in range(1, SLAB - 1):
        x3 = xn_ref[z - 1:z + 2, :]
        p = jnp.dot(w1_ref[:, 0:3], x3[:, 0:LI],
                    preferred_element_type=jnp.float32)
        p = p + jnp.dot(w1_ref[:, 3:6], x3[:, HP:HP + LI],
                        preferred_element_type=jnp.float32)
        p = p + jnp.dot(w1_ref[:, 6:9], x3[:, 2 * HP:2 * HP + LI],
                        preferred_element_type=jnp.float32)
        y = p[0:16, 0:LV] + p[16:32, 1:1 + LV] + p[32:48, 2:2 + LV]
        m = m2_ref[z:z + 1, 67:67 + LV]
        a1_ref[z, :, pl.ds(67, LV)] = (
            jnp.maximum(y + b1_ref[...], 0.0) * m)

    def conv2(z, _):
        x3 = a1_ref[pl.ds(z - 1, 3)]
        xt = jnp.concatenate(
            [x3[:, :, 0:LI].reshape(48, LI),
             x3[:, :, HP:HP + LI].reshape(48, LI),
             x3[:, :, 2 * HP:2 * HP + LI].reshape(48, LI)],
            axis=0).astype(jnp.bfloat16)
        p = jnp.dot(w2_ref[...], xt, preferred_element_type=jnp.float32)
        y = p[0:32, 0:LV] + p[32:64, 1:1 + LV] + p[64:96, 2:2 + LV]
        m = m3_ref[pl.ds(z, 1), :, 67:67 + LV].reshape(1, LV)
        y = jnp.maximum(y + b2_ref[...], 0.0) * m
        a2_ref[pl.ds(z, 1), :, pl.ds(67, LV)] = y.reshape(1, 32, LV)
        return 0

    def conv3(z, _):
        xt = a2_ref[pl.ds(z - 1, 3)].reshape(96, LP)
        p = jnp.dot(w3_ref[...], xt.astype(jnp.bfloat16),
                    preferred_element_type=jnp.float32)
        y = p[0:16, 67 + _OFF9[0]:67 + _OFF9[0] + LV]
        for t in range(1, 9):
            o = 67 + _OFF9[t]
            y = y + p[t * 16:(t + 1) * 16, o:o + LV]
        m = m3_ref[pl.ds(z, 1), :, 67:67 + LV].reshape(1, LV)
        y = jnp.maximum(y + b3_ref[...], 0.0) * m
        a3_ref[pl.ds(z, 1), :, pl.ds(67, LV)] = y.reshape(1, 16, LV)
        return 0

    jax.lax.fori_loop(2, SLAB - 2, conv2, 0)
    jax.lax.fori_loop(3, SLAB - 3, conv3, 0)

    for z in range(HALO, HALO + TILE_D):
        xt = a3_ref[z - 1:z + 2].reshape(48, LP)
        p4_ref[z - HALO] = jnp.dot(
            w4_ref[...], xt, preferred_element_type=jnp.float32)
    y = p4_ref[:, 0, 67 + _OFF9[0]:67 + _OFF9[0] + LV]
    for t in range(1, 9):
        o = 67 + _OFF9[t]
        y = y + p4_ref[:, t, o:o + LV]
    m = m2_ref[HALO:HALO + TILE_D, 67:67 + LV]
    y = jnp.maximum(y + b4_ref[...], 0.0) * m
    zpad = jnp.zeros((TILE_D, 67), jnp.float32)
    o_ref[0] = jnp.concatenate([zpad, y, zpad], axis=1) * 2.0 - 1.0


def kernel(inputTSDF, W1, b1, W2, b2, W3, b3, W4, b4):
    n = inputTSDF.shape[0]
    x = inputTSDF[:, 0]
    xp = jnp.pad(x, ((0, 0), (HALO, HALO), (1, 1), (1, 1)))
    xp = xp.reshape(n, D + 2 * HALO, LP)

    wm1 = W1[:, 0].transpose(3, 0, 2, 1).reshape(48, 9)
    wm2 = W2.transpose(4, 0, 3, 2, 1).reshape(96, 144)
    wm3 = W3.transpose(3, 4, 0, 2, 1).reshape(144, 96)
    wm4 = W4[0].transpose(2, 3, 1, 0).reshape(9, 48)

    grid = (n, D // TILE_D)
    full = lambda b, i: (0, 0)
    out = pl.pallas_call(
        _net_kernel,
        grid=grid,
        in_specs=[
            pl.BlockSpec((1, D + 2 * HALO, LP), lambda b, i: (b, 0, 0)),
            pl.BlockSpec((48, 9), full),
            pl.BlockSpec((16, 1), full),
            pl.BlockSpec((96, 144), full),
            pl.BlockSpec((32, 1), full),
            pl.BlockSpec((144, 96), full),
            pl.BlockSpec((16, 1), full),
            pl.BlockSpec((9, 48), full),
            pl.BlockSpec((1, 1), full),
        ],
        out_specs=pl.BlockSpec((1, TILE_D, LP), lambda b, i: (b, i, 0)),
        out_shape=jax.ShapeDtypeStruct((n, D, LP), jnp.float32),
        scratch_shapes=[
            pltpu.VMEM((SLAB, LP), jnp.float32),
            pltpu.VMEM((SLAB, LP), jnp.float32),
            pltpu.VMEM((SLAB, 1, LP), jnp.float32),
            pltpu.VMEM((SLAB, 16, LP), jnp.float32),
            pltpu.VMEM((SLAB, 32, LP), jnp.float32),
            pltpu.VMEM((SLAB, 16, LP), jnp.float32),
            pltpu.VMEM((TILE_D, 9, LP), jnp.float32),
        ],
        compiler_params=pltpu.CompilerParams(
            dimension_semantics=("parallel", "parallel")),
    )(xp, wm1, b1.reshape(16, 1), wm2.astype(jnp.bfloat16),
      b2.reshape(32, 1), wm3.astype(jnp.bfloat16), b3.reshape(16, 1),
      wm4, b4.reshape(1, 1))
    out = out.reshape(n, D, HP, HP)[:, :, 1:65, 1:65]
    return out[:, None]


# speedup vs baseline: 6.2822x; 1.0369x over previous
import functools

import jax
import jax.numpy as jnp
from jax.experimental import pallas as pl
from jax.experimental.pallas import tpu as pltpu

D = 64
HP = 66
LP = HP * HP
LI = LP - 2 * HP
LV = LP - 2 * HP - 2
TILE_D = 32
HALO = 4
SLAB = TILE_D + 2 * HALO

_OFF9 = (-67, -66, -65, -1, 0, 1, 65, 66, 67)


def _net_kernel(x_ref, w1_ref, b1_ref, w2_ref, b2_ref, w3_ref, b3_ref,
                w4_ref, b4_ref, o_ref,
                xn_ref, m2_ref, m3_ref, a1_ref, a2_ref, a3_ref, p4_ref):
    i = pl.program_id(1)

    for ref in (a1_ref, a2_ref, a3_ref):
        ref[:, :, 0:67] = jnp.zeros(ref.shape[:2] + (67,), jnp.float32)
        ref[:, :, LP - 67:LP] = jnp.zeros(ref.shape[:2] + (67,), jnp.float32)

    slab = x_ref[0, pl.ds(i * TILE_D, SLAB), :]
    mask = (jnp.abs(slab) != 0).astype(jnp.float32)
    xn_ref[...] = (slab / 2.0 + 0.5) * mask
    m2_ref[...] = mask
    for z in range(SLAB):
        m3_ref[z, :, :] = mask[z:z + 1, :]

    for z in range(1, SLAB - 1):
        x3 = xn_ref[z - 1:z + 2, :]
        p = jnp.dot(w1_ref[:, 0:3], x3[:, 0:LI],
                    preferred_element_type=jnp.float32)
        p = p + jnp.dot(w1_ref[:, 3:6], x3[:, HP:HP + LI],
                        preferred_element_type=jnp.float32)
        p = p + jnp.dot(w1_ref[:, 6:9], x3[:, 2 * HP:2 * HP + LI],
                        preferred_element_type=jnp.float32)
        y = p[0:16, 0:LV] + p[16:32, 1:1 + LV] + p[32:48, 2:2 + LV]
        m = m2_ref[z:z + 1, 67:67 + LV]
        a1_ref[z, :, pl.ds(67, LV)] = (
            jnp.maximum(y + b1_ref[...], 0.0) * m)

    def conv2(z, _):
        x3 = a1_ref[pl.ds(z - 1, 3)]
        xt = jnp.concatenate(
            [x3[:, :, 0:LI].reshape(48, LI),
             x3[:, :, HP:HP + LI].reshape(48, LI),
             x3[:, :, 2 * HP:2 * HP + LI].reshape(48, LI)],
            axis=0)
        p = jnp.dot(w2_ref[...], xt, preferred_element_type=jnp.float32)
        y = p[0:32, 0:LV] + p[32:64, 1:1 + LV] + p[64:96, 2:2 + LV]
        m = m3_ref[pl.ds(z, 1), :, 67:67 + LV].reshape(1, LV)
        y = jnp.maximum(y + b2_ref[...], 0.0) * m
        a2_ref[pl.ds(z, 1), :, pl.ds(67, LV)] = y.reshape(1, 32, LV)
        return 0

    def conv3(z, _):
        xt = a2_ref[pl.ds(z - 1, 3)].reshape(96, LP)
        p = jnp.dot(w3_ref[...], xt, preferred_element_type=jnp.float32)
        y = p[0:16, 67 + _OFF9[0]:67 + _OFF9[0] + LV]
        for t in range(1, 9):
            o = 67 + _OFF9[t]
            y = y + p[t * 16:(t + 1) * 16, o:o + LV]
        m = m3_ref[pl.ds(z, 1), :, 67:67 + LV].reshape(1, LV)
        y = jnp.maximum(y + b3_ref[...], 0.0) * m
        a3_ref[pl.ds(z, 1), :, pl.ds(67, LV)] = y.reshape(1, 16, LV)
        return 0

    jax.lax.fori_loop(2, SLAB - 2, conv2, 0)
    jax.lax.fori_loop(3, SLAB - 3, conv3, 0)

    HT = TILE_D // 2
    zpad = jnp.zeros((HT, 67), jnp.float32)
    for h in range(2):
        z0 = HALO + h * HT
        for z in range(z0, z0 + HT):
            xt = a3_ref[z - 1:z + 2].reshape(48, LP)
            p4_ref[z - z0] = jnp.dot(
                w4_ref[...], xt, preferred_element_type=jnp.float32)
        y = p4_ref[:, 0, 67 + _OFF9[0]:67 + _OFF9[0] + LV]
        for t in range(1, 9):
            o = 67 + _OFF9[t]
            y = y + p4_ref[:, t, o:o + LV]
        m = m2_ref[z0:z0 + HT, 67:67 + LV]
        y = jnp.maximum(y + b4_ref[...], 0.0) * m
        o_ref[0, h * HT:(h + 1) * HT, :] = (
            jnp.concatenate([zpad, y, zpad], axis=1) * 2.0 - 1.0)


def kernel(inputTSDF, W1, b1, W2, b2, W3, b3, W4, b4):
    n = inputTSDF.shape[0]
    x = inputTSDF[:, 0]
    xp = jnp.pad(x, ((0, 0), (HALO, HALO), (1, 1), (1, 1)))
    xp = xp.reshape(n, D + 2 * HALO, LP)

    wm1 = W1[:, 0].transpose(3, 0, 2, 1).reshape(48, 9)
    wm2 = W2.transpose(4, 0, 3, 2, 1).reshape(96, 144)
    wm3 = W3.transpose(3, 4, 0, 2, 1).reshape(144, 96)
    wm4 = W4[0].transpose(2, 3, 1, 0).reshape(9, 48)

    grid = (n, D // TILE_D)
    full = lambda b, i: (0, 0)
    out = pl.pallas_call(
        _net_kernel,
        grid=grid,
        in_specs=[
            pl.BlockSpec((1, D + 2 * HALO, LP), lambda b, i: (b, 0, 0)),
            pl.BlockSpec((48, 9), full),
            pl.BlockSpec((16, 1), full),
            pl.BlockSpec((96, 144), full),
            pl.BlockSpec((32, 1), full),
            pl.BlockSpec((144, 96), full),
            pl.BlockSpec((16, 1), full),
            pl.BlockSpec((9, 48), full),
            pl.BlockSpec((1, 1), full),
        ],
        out_specs=pl.BlockSpec((1, TILE_D, LP), lambda b, i: (b, i, 0)),
        out_shape=jax.ShapeDtypeStruct((n, D, LP), jnp.float32),
        scratch_shapes=[
            pltpu.VMEM((SLAB, LP), jnp.float32),
            pltpu.VMEM((SLAB, LP), jnp.float32),
            pltpu.VMEM((SLAB, 1, LP), jnp.float32),
            pltpu.VMEM((SLAB, 16, LP), jnp.float32),
            pltpu.VMEM((SLAB, 32, LP), jnp.float32),
            pltpu.VMEM((SLAB, 16, LP), jnp.float32),
            pltpu.VMEM((TILE_D // 2, 9, LP), jnp.float32),
        ],
        compiler_params=pltpu.CompilerParams(
            dimension_semantics=("parallel", "parallel")),
    )(xp, wm1, b1.reshape(16, 1), wm2, b2.reshape(32, 1),
      wm3, b3.reshape(16, 1), wm4, b4.reshape(1, 1))
    out = out.reshape(n, D, HP, HP)[:, :, 1:65, 1:65]
    return out[:, None]


# speedup vs baseline: 6.2909x; 1.0014x over previous
import jax
import jax.numpy as jnp
from jax.experimental import pallas as pl
from jax.experimental.pallas import tpu as pltpu

D = 64
HP = 66
LP = HP * HP
LI = LP - 2 * HP
LV = LP - 2 * HP - 2
TILE_D = 32
HALO = 4
SLAB = TILE_D + 2 * HALO

_OFF9 = (-67, -66, -65, -1, 0, 1, 65, 66, 67)


def _net_kernel(x_ref, w1_ref, b1_ref, w2_ref, b2_ref, w3_ref, b3_ref,
                w4_ref, b4_ref, o_ref,
                xn_ref, m2_ref, m3_ref, a1_ref, a2_ref, a3_ref, p4_ref):
    i = pl.program_id(1)

    for ref in (a1_ref, a2_ref, a3_ref):
        ref[:, :, 0:67] = jnp.zeros(ref.shape[:2] + (67,), jnp.float32)
        ref[:, :, LP - 67:LP] = jnp.zeros(ref.shape[:2] + (67,), jnp.float32)

    slab = x_ref[0, pl.ds(i * TILE_D, SLAB), :]
    mask = (jnp.abs(slab) != 0).astype(jnp.float32)
    xn_ref[...] = (slab / 2.0 + 0.5) * mask
    m2_ref[...] = mask
    for z in range(SLAB):
        m3_ref[z, :, :] = mask[z:z + 1, :]

    for z in range(1, SLAB - 1):
        x3 = xn_ref[z - 1:z + 2, :]
        p = jnp.dot(w1_ref[:, 0:3], x3[:, 0:LI],
                    preferred_element_type=jnp.float32)
        p = p + jnp.dot(w1_ref[:, 3:6], x3[:, HP:HP + LI],
                        preferred_element_type=jnp.float32)
        p = p + jnp.dot(w1_ref[:, 6:9], x3[:, 2 * HP:2 * HP + LI],
                        preferred_element_type=jnp.float32)
        y = p[0:16, 0:LV] + p[16:32, 1:1 + LV] + p[32:48, 2:2 + LV]
        m = m2_ref[z:z + 1, 67:67 + LV]
        a1_ref[z, :, pl.ds(67, LV)] = (
            jnp.maximum(y + b1_ref[...], 0.0) * m)

    def conv2(z, _):
        x3 = a1_ref[pl.ds(z - 1, 3)]
        xt = jnp.concatenate(
            [x3[:, :, 0:LI].reshape(48, LI),
             x3[:, :, HP:HP + LI].reshape(48, LI),
             x3[:, :, 2 * HP:2 * HP + LI].reshape(48, LI)],
            axis=0)
        p = jnp.dot(w2_ref[...], xt, preferred_element_type=jnp.float32)
        y = p[0:32, 0:LV] + p[32:64, 1:1 + LV] + p[64:96, 2:2 + LV]
        m = m3_ref[pl.ds(z, 1), :, 67:67 + LV].reshape(1, LV)
        y = jnp.maximum(y + b2_ref[...], 0.0) * m
        a2_ref[pl.ds(z, 1), :, pl.ds(67, LV)] = y.reshape(1, 32, LV)
        return 0

    def conv3(z, _):
        xt = a2_ref[pl.ds(z - 1, 3)].reshape(96, LP)
        p = jnp.dot(w3_ref[...], xt, preferred_element_type=jnp.float32)
        y = p[0:16, 67 + _OFF9[0]:67 + _OFF9[0] + LV]
        for t in range(1, 9):
            o = 67 + _OFF9[t]
            y = y + p[t * 16:(t + 1) * 16, o:o + LV]
        m = m3_ref[pl.ds(z, 1), :, 67:67 + LV].reshape(1, LV)
        y = jnp.maximum(y + b3_ref[...], 0.0) * m
        a3_ref[pl.ds(z, 1), :, pl.ds(67, LV)] = y.reshape(1, 16, LV)
        return 0

    jax.lax.fori_loop(2, SLAB - 2, conv2, 0)
    jax.lax.fori_loop(3, SLAB - 3, conv3, 0)

    HT = TILE_D // 2
    zpad = jnp.zeros((HT, 67), jnp.float32)
    for h in range(2):
        z0 = HALO + h * HT
        for z in range(z0, z0 + HT):
            xt = a3_ref[z - 1:z + 2].reshape(48, LP)
            p4_ref[z - z0] = jnp.dot(
                w4_ref[...], xt, preferred_element_type=jnp.float32)
        y = p4_ref[:, 0, 67 + _OFF9[0]:67 + _OFF9[0] + LV]
        for t in range(1, 9):
            o = 67 + _OFF9[t]
            y = y + p4_ref[:, t, o:o + LV]
        m = m2_ref[z0:z0 + HT, 67:67 + LV]
        y = jnp.maximum(y + b4_ref[...], 0.0) * m
        o_ref[0, h * HT:(h + 1) * HT, :] = (
            jnp.concatenate([zpad, y, zpad], axis=1) * 2.0 - 1.0)


def kernel(inputTSDF, W1, b1, W2, b2, W3, b3, W4, b4):
    n = inputTSDF.shape[0]
    x = inputTSDF[:, 0]
    xp = jnp.pad(x, ((0, 0), (HALO, HALO), (1, 1), (1, 1)))
    xp = xp.reshape(n, D + 2 * HALO, LP)

    wm1 = W1[:, 0].transpose(3, 0, 2, 1).reshape(48, 9)
    wm2 = W2.transpose(4, 0, 3, 2, 1).reshape(96, 144)
    wm3 = W3.transpose(3, 4, 0, 2, 1).reshape(144, 96)
    wm4 = W4[0].transpose(2, 3, 1, 0).reshape(9, 48)

    grid = (n, D // TILE_D)
    full = lambda b, i: (0, 0)
    out = pl.pallas_call(
        _net_kernel,
        grid=grid,
        in_specs=[
            pl.BlockSpec((1, D + 2 * HALO, LP), lambda b, i: (b, 0, 0)),
            pl.BlockSpec((48, 9), full),
            pl.BlockSpec((16, 1), full),
            pl.BlockSpec((96, 144), full),
            pl.BlockSpec((32, 1), full),
            pl.BlockSpec((144, 96), full),
            pl.BlockSpec((16, 1), full),
            pl.BlockSpec((9, 48), full),
            pl.BlockSpec((1, 1), full),
        ],
        out_specs=pl.BlockSpec((1, TILE_D, LP), lambda b, i: (b, i, 0)),
        out_shape=jax.ShapeDtypeStruct((n, D, LP), jnp.float32),
        scratch_shapes=[
            pltpu.VMEM((SLAB, LP), jnp.float32),
            pltpu.VMEM((SLAB, LP), jnp.float32),
            pltpu.VMEM((SLAB, 1, LP), jnp.float32),
            pltpu.VMEM((SLAB, 16, LP), jnp.float32),
            pltpu.VMEM((SLAB, 32, LP), jnp.float32),
            pltpu.VMEM((SLAB, 16, LP), jnp.float32),
            pltpu.VMEM((TILE_D // 2, 9, LP), jnp.float32),
        ],
        compiler_params=pltpu.CompilerParams(
            dimension_semantics=("parallel", "parallel")),
    )(xp, wm1, b1.reshape(16, 1), wm2, b2.reshape(32, 1),
      wm3, b3.reshape(16, 1), wm4, b4.reshape(1, 1))
    out = out.reshape(n, D, HP, HP)[:, :, 1:65, 1:65]
    return out[:, None]
